# initial kernel scaffold (unmeasured)
import jax
import jax.numpy as jnp
from jax import lax
from jax.experimental import pallas as pl
from jax.experimental.pallas import tpu as pltpu


def kernel(ids, E):
    v_per, d = E.shape
    z = lax.axis_index("z")
    local = ids.astype(jnp.int32) - z * v_per
    mask = (local >= 0) & (local < v_per)
    safe = jnp.where(mask, local, 0)
    partial = jnp.where(
        mask[:, None], jnp.take(E, safe, axis=0), jnp.float32(0.0)
    ).astype(jnp.float32)

    def body(p_ref, out_ref, comm_ref, send_sem, recv_sem):
        my_x = lax.axis_index("x")
        my_y = lax.axis_index("y")
        my_z = lax.axis_index("z")
        partner = (my_x, my_y, 1 - my_z)

        barrier = pltpu.get_barrier_semaphore()
        pl.semaphore_signal(
            barrier, inc=1, device_id=partner,
            device_id_type=pl.DeviceIdType.MESH,
        )
        pl.semaphore_wait(barrier, 1)

        rdma = pltpu.make_async_remote_copy(
            src_ref=p_ref,
            dst_ref=comm_ref,
            send_sem=send_sem,
            recv_sem=recv_sem,
            device_id=partner,
            device_id_type=pl.DeviceIdType.MESH,
        )
        rdma.start()
        rdma.wait()

        out_ref[...] = p_ref[...] + comm_ref[...]

    return pl.pallas_call(
        body,
        out_shape=jax.ShapeDtypeStruct(partial.shape, jnp.float32),
        in_specs=[pl.BlockSpec(memory_space=pltpu.VMEM)],
        out_specs=pl.BlockSpec(memory_space=pltpu.VMEM),
        scratch_shapes=[
            pltpu.VMEM(partial.shape, jnp.float32),
            pltpu.SemaphoreType.DMA,
            pltpu.SemaphoreType.DMA,
        ],
        compiler_params=pltpu.CompilerParams(collective_id=0),
    )(partial)


# baseline (device time: 86561 ns/iter reference)
import jax
import jax.numpy as jnp
from jax import lax
from jax.experimental import pallas as pl
from jax.experimental.pallas import tpu as pltpu


def kernel(ids, E):
    v_per, d = E.shape
    z = lax.axis_index("z")
    local = ids.astype(jnp.int32) - z * v_per
    mask = (local >= 0) & (local < v_per)
    safe = jnp.where(mask, local, 0)
    partial = jnp.where(
        mask[:, None], jnp.take(E, safe, axis=0), jnp.float32(0.0)
    ).astype(jnp.float32)

    def body(p_ref, out_ref, comm_ref, send_sem, recv_sem):
        my_x = lax.axis_index("x")
        my_y = lax.axis_index("y")
        my_z = lax.axis_index("z")
        partner = (my_x, my_y, 1 - my_z)

        barrier = pltpu.get_barrier_semaphore()
        pl.semaphore_signal(
            barrier, inc=1, device_id=partner,
            device_id_type=pl.DeviceIdType.MESH,
        )
        pl.semaphore_wait(barrier, 1)

        rdma = pltpu.make_async_remote_copy(
            src_ref=p_ref,
            dst_ref=comm_ref,
            send_sem=send_sem,
            recv_sem=recv_sem,
            device_id=partner,
            device_id_type=pl.DeviceIdType.MESH,
        )
        rdma.start()
        rdma.wait()

        out_ref[...] = p_ref[...] + comm_ref[...]

    return pl.pallas_call(
        body,
        out_shape=jax.ShapeDtypeStruct(partial.shape, jnp.float32),
        in_specs=[pl.BlockSpec(memory_space=pltpu.VMEM)],
        out_specs=pl.BlockSpec(memory_space=pltpu.VMEM),
        scratch_shapes=[
            pltpu.VMEM(partial.shape, jnp.float32),
            pltpu.SemaphoreType.DMA,
            pltpu.SemaphoreType.DMA,
        ],
        compiler_params=pltpu.CompilerParams(collective_id=11),
    )(partial)


# device time: 44159 ns/iter; 1.9602x vs baseline; 1.9602x over previous
import jax
import jax.numpy as jnp
from jax import lax
from jax.experimental import pallas as pl
from jax.experimental.pallas import tpu as pltpu


def kernel(ids, E):
    v_per, d = E.shape
    t_len = ids.shape[0]
    z = lax.axis_index("z")
    local = ids.astype(jnp.int32) - z * v_per
    mask = (local >= 0) & (local < v_per)
    n_own = jnp.sum(mask.astype(jnp.int32))
    order = jnp.argsort(jnp.logical_not(mask), stable=True).astype(jnp.int32)
    own_idx = jnp.clip(local[order], 0, v_per - 1).astype(jnp.int32)
    counts = jnp.stack([n_own, jnp.int32(t_len) - n_own])

    def body(tok_ref, idx_ref, cnt_ref, e_ref, out_ref,
             local_sem, send_sem, recv_sem):
        my_x = lax.axis_index("x")
        my_y = lax.axis_index("y")
        my_z = lax.axis_index("z")
        partner = (my_x, my_y, 1 - my_z)

        barrier = pltpu.get_barrier_semaphore()
        pl.semaphore_signal(
            barrier, inc=1, device_id=partner,
            device_id_type=pl.DeviceIdType.MESH,
        )
        pl.semaphore_wait(barrier, 1)

        n_mine = cnt_ref[0]
        n_peer = cnt_ref[1]

        def issue(i, carry):
            t = tok_ref[i]
            r = idx_ref[i]
            pltpu.make_async_copy(
                e_ref.at[pl.ds(r, 1), :],
                out_ref.at[pl.ds(t, 1), :],
                local_sem,
            ).start()
            pltpu.make_async_remote_copy(
                src_ref=e_ref.at[pl.ds(r, 1), :],
                dst_ref=out_ref.at[pl.ds(t, 1), :],
                send_sem=send_sem,
                recv_sem=recv_sem,
                device_id=partner,
                device_id_type=pl.DeviceIdType.MESH,
            ).start()
            return carry

        lax.fori_loop(0, n_mine, issue, 0)

        def dummy_rdma():
            return pltpu.make_async_remote_copy(
                src_ref=e_ref.at[pl.ds(0, 1), :],
                dst_ref=out_ref.at[pl.ds(0, 1), :],
                send_sem=send_sem,
                recv_sem=recv_sem,
                device_id=partner,
                device_id_type=pl.DeviceIdType.MESH,
            )

        def wait_local(i, c):
            pltpu.make_async_copy(
                e_ref.at[pl.ds(0, 1), :],
                out_ref.at[pl.ds(0, 1), :],
                local_sem,
            ).wait()
            return c

        def wait_send(i, c):
            dummy_rdma().wait_send()
            return c

        def wait_recv(i, c):
            dummy_rdma().wait_recv()
            return c

        lax.fori_loop(0, n_mine, wait_local, 0)
        lax.fori_loop(0, n_mine, wait_send, 0)
        lax.fori_loop(0, n_peer, wait_recv, 0)

    return pl.pallas_call(
        body,
        out_shape=jax.ShapeDtypeStruct((t_len, d), jnp.float32),
        in_specs=[
            pl.BlockSpec(memory_space=pltpu.SMEM),
            pl.BlockSpec(memory_space=pltpu.SMEM),
            pl.BlockSpec(memory_space=pltpu.SMEM),
            pl.BlockSpec(memory_space=pl.ANY),
        ],
        out_specs=pl.BlockSpec(memory_space=pltpu.VMEM),
        scratch_shapes=[
            pltpu.SemaphoreType.DMA,
            pltpu.SemaphoreType.DMA,
            pltpu.SemaphoreType.DMA,
        ],
        compiler_params=pltpu.CompilerParams(collective_id=11),
    )(order, own_idx, counts, E)
